# Initial kernel scaffold; baseline (speedup 1.0000x reference)
import jax
import jax.numpy as jnp
from jax.experimental import pallas as pl


def kernel(f_list, tables, W, b):
    raise NotImplementedError("write your pallas kernel here")


# speedup vs baseline: 14.6672x; 14.6672x over previous
import functools

import jax
import jax.numpy as jnp
from jax import lax
from jax.experimental import pallas as pl
from jax.experimental.pallas import tpu as pltpu
from jax.experimental.pallas import tpu_sc as plsc

NC = 2
NS = 16
CH = 128


def _sc_gather(tab_flat, fl_flat, F, V, D, B):
    NW = NC * NS
    n_tot = B * F
    n_per_w = n_tot // NW
    assert n_per_w * NW == n_tot and n_per_w % CH == 0
    n_ch = n_per_w // CH
    assert n_ch % 2 == 0
    n_pairs = n_ch // 2

    mesh = plsc.VectorSubcoreMesh(core_axis_name="c", subcore_axis_name="s")

    @functools.partial(
        pl.kernel,
        mesh=mesh,
        out_type=jax.ShapeDtypeStruct((n_tot, D), jnp.float32),
        scratch_types=[
            pltpu.VMEM((n_per_w,), jnp.int32),
            pltpu.VMEM((CH, D), jnp.float32),
            pltpu.VMEM((CH, D), jnp.float32),
            pltpu.SemaphoreType.DMA,
            pltpu.SemaphoreType.DMA,
            pltpu.SemaphoreType.DMA,
            pltpu.SemaphoreType.DMA,
        ],
    )
    def gather_kernel(tab_hbm, fl_hbm, out_hbm, idx_v, rows0, rows1,
                      gs0, gs1, ss0, ss1):
        wid = lax.axis_index("s") * NC + lax.axis_index("c")
        base = wid * n_per_w

        pltpu.sync_copy(fl_hbm.at[pl.ds(base, n_per_w)], idx_v)

        def idx_body(i, carry):
            j0 = base + i * 16
            jv = jnp.full((16,), j0, jnp.int32) + lax.iota(jnp.int32, 16)
            off = (jv % F) * V
            idx_v[pl.ds(i * 16, 16)] = idx_v[pl.ds(i * 16, 16)] + off
            return carry

        lax.fori_loop(0, n_per_w // 16, idx_body, 0)

        def g_src(c):
            return tab_hbm.at[idx_v.at[pl.ds(c * CH, CH)]]

        def s_dst(c):
            return out_hbm.at[pl.ds(base + c * CH, CH)]

        pltpu.async_copy(g_src(0), rows0, gs0)
        pltpu.async_copy(g_src(1), rows1, gs1)

        def pair_body(p, carry):
            c0 = p * 2
            c1 = c0 + 1
            pltpu.make_async_copy(g_src(c0), rows0, gs0).wait()
            pltpu.async_copy(rows0, s_dst(c0), ss0)
            pltpu.make_async_copy(g_src(c1), rows1, gs1).wait()
            pltpu.async_copy(rows1, s_dst(c1), ss1)
            pltpu.make_async_copy(rows0, s_dst(c0), ss0).wait()

            @pl.when(p < n_pairs - 1)
            def _():
                pltpu.async_copy(g_src(c0 + 2), rows0, gs0)

            pltpu.make_async_copy(rows1, s_dst(c1), ss1).wait()

            @pl.when(p < n_pairs - 1)
            def _():
                pltpu.async_copy(g_src(c1 + 2), rows1, gs1)

            return carry

        lax.fori_loop(0, n_pairs, pair_body, 0)

    return gather_kernel(tab_flat, fl_flat)


def _tc_linear_relu(emb, w_bf16, bias, B, K, D):
    BM = 256

    def mm_kernel(e_ref, w_ref, b_ref, o_ref):
        e = e_ref[...].astype(jnp.bfloat16)
        acc = jnp.dot(e, w_ref[...], preferred_element_type=jnp.float32)
        o_ref[...] = jnp.maximum(acc + b_ref[...], 0.0)

    return pl.pallas_call(
        mm_kernel,
        grid=(B // BM,),
        in_specs=[
            pl.BlockSpec((BM, K), lambda i: (i, 0)),
            pl.BlockSpec((K, D), lambda i: (0, 0)),
            pl.BlockSpec((1, D), lambda i: (0, 0)),
        ],
        out_specs=pl.BlockSpec((BM, D), lambda i: (i, 0)),
        out_shape=jax.ShapeDtypeStruct((B, D), jnp.float32),
    )(emb, w_bf16, bias)


def kernel(f_list, tables, W, b):
    F, V, D = tables.shape
    B = f_list.shape[0]
    tab_flat = tables.reshape(F * V, D)
    fl_flat = f_list.reshape(B * F).astype(jnp.int32)
    emb_flat = _sc_gather(tab_flat, fl_flat, F, V, D, B)
    emb = emb_flat.reshape(B, F * D)
    return _tc_linear_relu(emb, W.astype(jnp.bfloat16), b.reshape(1, D),
                           B, F * D, D)


# speedup vs baseline: 26.3971x; 1.7997x over previous
import functools

import jax
import jax.numpy as jnp
from jax import lax
from jax.experimental import pallas as pl
from jax.experimental.pallas import tpu as pltpu
from jax.experimental.pallas import tpu_sc as plsc

NC = 2
NS = 16
CH = 128


def _sc_gather(tab_flat, fl_flat, F, V, D, B):
    b_shift = B.bit_length() - 1
    assert (1 << b_shift) == B and B % CH == 0
    NW = NC * NS
    n_tot = B * F
    n_per_w = n_tot // NW
    assert n_per_w * NW == n_tot and n_per_w % CH == 0
    n_ch = n_per_w // CH
    assert n_ch % 2 == 0
    n_pairs = n_ch // 2

    mesh = plsc.VectorSubcoreMesh(core_axis_name="c", subcore_axis_name="s")

    @functools.partial(
        pl.kernel,
        mesh=mesh,
        out_type=jax.ShapeDtypeStruct((n_tot, D), jnp.float32),
        scratch_types=[
            pltpu.VMEM((n_per_w,), jnp.int32),
            pltpu.VMEM((CH, D), jnp.float32),
            pltpu.VMEM((CH, D), jnp.float32),
            pltpu.SemaphoreType.DMA,
            pltpu.SemaphoreType.DMA,
            pltpu.SemaphoreType.DMA,
            pltpu.SemaphoreType.DMA,
        ],
    )
    def gather_kernel(tab_hbm, fl_hbm, out_hbm, idx_v, rows0, rows1,
                      gs0, gs1, ss0, ss1):
        wid = lax.axis_index("s") * NC + lax.axis_index("c")
        base = wid * n_per_w

        pltpu.sync_copy(fl_hbm.at[pl.ds(base, n_per_w)], idx_v)

        def idx_body(i, carry):
            j0 = base + i * 16
            jv = jnp.full((16,), j0, jnp.int32) + lax.iota(jnp.int32, 16)
            off = lax.shift_right_logical(jv, b_shift) * V
            idx_v[pl.ds(i * 16, 16)] = idx_v[pl.ds(i * 16, 16)] + off
            return carry

        lax.fori_loop(0, n_per_w // 16, idx_body, 0)

        def g_src(c):
            return tab_hbm.at[idx_v.at[pl.ds(c * CH, CH)]]

        def s_dst(c):
            return out_hbm.at[pl.ds(base + c * CH, CH)]

        pltpu.async_copy(g_src(0), rows0, gs0)
        pltpu.async_copy(g_src(1), rows1, gs1)

        def pair_body(p, carry):
            c0 = p * 2
            c1 = c0 + 1
            pltpu.make_async_copy(g_src(c0), rows0, gs0).wait()
            pltpu.async_copy(rows0, s_dst(c0), ss0)
            pltpu.make_async_copy(g_src(c1), rows1, gs1).wait()
            pltpu.async_copy(rows1, s_dst(c1), ss1)
            pltpu.make_async_copy(rows0, s_dst(c0), ss0).wait()

            @pl.when(p < n_pairs - 1)
            def _():
                pltpu.async_copy(g_src(c0 + 2), rows0, gs0)

            pltpu.make_async_copy(rows1, s_dst(c1), ss1).wait()

            @pl.when(p < n_pairs - 1)
            def _():
                pltpu.async_copy(g_src(c1 + 2), rows1, gs1)

            return carry

        lax.fori_loop(0, n_pairs, pair_body, 0)

    return gather_kernel(tab_flat, fl_flat)


def _tc_linear_relu(emb3, w_bf16, bias, F, B, D, DOUT):
    BM = 256

    def mm_kernel(e_ref, w_ref, b_ref, o_ref):
        acc = jnp.zeros((BM, DOUT), jnp.float32)
        for k in range(F // 2):
            e2 = jnp.concatenate(
                [e_ref[2 * k].astype(jnp.bfloat16),
                 e_ref[2 * k + 1].astype(jnp.bfloat16)], axis=-1)
            acc = acc + jnp.dot(e2, w_ref[pl.ds(2 * k * D, 2 * D), :],
                                preferred_element_type=jnp.float32)
        if F % 2:
            acc = acc + jnp.dot(e_ref[F - 1].astype(jnp.bfloat16),
                                w_ref[pl.ds((F - 1) * D, D), :],
                                preferred_element_type=jnp.float32)
        o_ref[...] = jnp.maximum(acc + b_ref[...], 0.0)

    return pl.pallas_call(
        mm_kernel,
        grid=(B // BM,),
        in_specs=[
            pl.BlockSpec((F, BM, D), lambda i: (0, i, 0)),
            pl.BlockSpec((F * D, DOUT), lambda i: (0, 0)),
            pl.BlockSpec((1, DOUT), lambda i: (0, 0)),
        ],
        out_specs=pl.BlockSpec((BM, DOUT), lambda i: (i, 0)),
        out_shape=jax.ShapeDtypeStruct((B, DOUT), jnp.float32),
    )(emb3, w_bf16, bias)


def kernel(f_list, tables, W, b):
    F, V, D = tables.shape
    B = f_list.shape[0]
    DOUT = W.shape[1]
    tab_flat = tables.reshape(F * V, D)
    fl_fm = f_list.T.reshape(B * F).astype(jnp.int32)
    emb_fm = _sc_gather(tab_flat, fl_fm, F, V, D, B)
    emb3 = emb_fm.reshape(F, B, D)
    return _tc_linear_relu(emb3, W.astype(jnp.bfloat16), b.reshape(1, DOUT),
                           F, B, D, DOUT)


# speedup vs baseline: 26.6896x; 1.0111x over previous
import functools

import jax
import jax.numpy as jnp
from jax import lax
from jax.experimental import pallas as pl
from jax.experimental.pallas import tpu as pltpu
from jax.experimental.pallas import tpu_sc as plsc

NC = 2
NS = 16
CH = 128


def _sc_gather(tab_flat, fl_flat, F, V, D, B):
    b_shift = B.bit_length() - 1
    assert (1 << b_shift) == B and B % CH == 0
    NW = NC * NS
    n_tot = B * F
    n_per_w = n_tot // NW
    assert n_per_w * NW == n_tot and n_per_w % CH == 0
    n_ch = n_per_w // CH
    assert n_ch % 2 == 0
    n_pairs = n_ch // 2

    mesh = plsc.VectorSubcoreMesh(core_axis_name="c", subcore_axis_name="s")

    @functools.partial(
        pl.kernel,
        mesh=mesh,
        out_type=jax.ShapeDtypeStruct((n_tot, D), jnp.float32),
        scratch_types=[
            pltpu.VMEM((n_per_w,), jnp.int32),
            pltpu.VMEM((CH, D), jnp.float32),
            pltpu.VMEM((CH, D), jnp.float32),
            pltpu.SemaphoreType.DMA,
            pltpu.SemaphoreType.DMA,
            pltpu.SemaphoreType.DMA,
            pltpu.SemaphoreType.DMA,
        ],
    )
    def gather_kernel(tab_hbm, fl_hbm, out_hbm, idx_v, rows0, rows1,
                      gs0, gs1, ss0, ss1):
        wid = lax.axis_index("s") * NC + lax.axis_index("c")
        base = wid * n_per_w

        pltpu.sync_copy(fl_hbm.at[pl.ds(base, n_per_w)], idx_v)

        def idx_body(i, carry):
            j0 = base + i * 16
            jv = jnp.full((16,), j0, jnp.int32) + lax.iota(jnp.int32, 16)
            off = lax.shift_right_logical(jv, b_shift) * V
            idx_v[pl.ds(i * 16, 16)] = idx_v[pl.ds(i * 16, 16)] + off
            return carry

        lax.fori_loop(0, n_per_w // 16, idx_body, 0)

        def g_src(c):
            return tab_hbm.at[idx_v.at[pl.ds(c * CH, CH)]]

        def s_dst(c):
            return out_hbm.at[pl.ds(base + c * CH, CH)]

        pltpu.async_copy(g_src(0), rows0, gs0)
        pltpu.async_copy(g_src(1), rows1, gs1)

        def pair_body(p, carry):
            c0 = p * 2
            c1 = c0 + 1
            pltpu.make_async_copy(g_src(c0), rows0, gs0).wait()
            pltpu.async_copy(rows0, s_dst(c0), ss0)
            pltpu.make_async_copy(g_src(c1), rows1, gs1).wait()
            pltpu.async_copy(rows1, s_dst(c1), ss1)
            pltpu.make_async_copy(rows0, s_dst(c0), ss0).wait()

            @pl.when(p < n_pairs - 1)
            def _():
                pltpu.async_copy(g_src(c0 + 2), rows0, gs0)

            pltpu.make_async_copy(rows1, s_dst(c1), ss1).wait()

            @pl.when(p < n_pairs - 1)
            def _():
                pltpu.async_copy(g_src(c1 + 2), rows1, gs1)

            return carry

        lax.fori_loop(0, n_pairs, pair_body, 0)

    return gather_kernel(tab_flat, fl_flat)


def _tc_linear_relu(emb3, w_bf16, bias, F, B, D, DOUT):
    BM = 256

    def mm_kernel(e_ref, w_ref, b_ref, o_ref):
        acc = jnp.zeros((BM, DOUT), jnp.float32)
        for k in range(F // 2):
            e2 = jnp.concatenate(
                [e_ref[2 * k].astype(jnp.bfloat16),
                 e_ref[2 * k + 1].astype(jnp.bfloat16)], axis=-1)
            acc = acc + jnp.dot(e2, w_ref[pl.ds(2 * k * D, 2 * D), :],
                                preferred_element_type=jnp.float32)
        if F % 2:
            acc = acc + jnp.dot(e_ref[F - 1].astype(jnp.bfloat16),
                                w_ref[pl.ds((F - 1) * D, D), :],
                                preferred_element_type=jnp.float32)
        o_ref[...] = jnp.maximum(acc + b_ref[...], 0.0)

    return pl.pallas_call(
        mm_kernel,
        grid=(B // BM,),
        in_specs=[
            pl.BlockSpec((F, BM, D), lambda i: (0, i, 0)),
            pl.BlockSpec((F * D, DOUT), lambda i: (0, 0)),
            pl.BlockSpec((1, DOUT), lambda i: (0, 0)),
        ],
        out_specs=pl.BlockSpec((BM, DOUT), lambda i: (i, 0)),
        out_shape=jax.ShapeDtypeStruct((B, DOUT), jnp.float32),
    )(emb3, w_bf16, bias)


NCHUNK = 4


def kernel(f_list, tables, W, b):
    F, V, D = tables.shape
    B = f_list.shape[0]
    DOUT = W.shape[1]
    tab_flat = tables.reshape(F * V, D)
    fl_t = f_list.T.astype(jnp.int32)
    w_bf16 = W.astype(jnp.bfloat16)
    bias2 = b.reshape(1, DOUT)
    bc = B // NCHUNK
    outs = []
    for c in range(NCHUNK):
        fl_c = fl_t[:, c * bc:(c + 1) * bc].reshape(F * bc)
        emb_c = _sc_gather(tab_flat, fl_c, F, V, D, bc)
        outs.append(_tc_linear_relu(emb_c.reshape(F, bc, D), w_bf16, bias2,
                                    F, bc, D, DOUT))
    return jnp.concatenate(outs, axis=0)


# speedup vs baseline: 27.5862x; 1.0336x over previous
import functools

import jax
import jax.numpy as jnp
from jax import lax
from jax.experimental import pallas as pl
from jax.experimental.pallas import tpu as pltpu
from jax.experimental.pallas import tpu_sc as plsc

NC = 2
NS = 16
CH = 128
NBUF = 4
NCHUNK = 2


def _sc_gather(tab_flat, fl_fm, F, V, D, B):
    b_shift = B.bit_length() - 1
    assert (1 << b_shift) == B and B % CH == 0
    NW = NC * NS
    n_tot = B * F
    n_per_w = n_tot // NW
    assert n_per_w * NW == n_tot and n_per_w % CH == 0
    n_ch = n_per_w // CH
    assert n_ch % NBUF == 0
    n_groups = n_ch // NBUF

    mesh = plsc.VectorSubcoreMesh(core_axis_name="c", subcore_axis_name="s")

    @functools.partial(
        pl.kernel,
        mesh=mesh,
        out_type=jax.ShapeDtypeStruct((n_tot, D), jnp.float32),
        scratch_types=(
            [pltpu.VMEM((n_per_w,), jnp.int32)]
            + [pltpu.VMEM((CH, D), jnp.float32) for _ in range(NBUF)]
            + [pltpu.SemaphoreType.DMA for _ in range(2 * NBUF)]
        ),
    )
    def gather_kernel(tab_hbm, fl_hbm, out_hbm, idx_v, *bufs_and_sems):
        rows = bufs_and_sems[:NBUF]
        gsem = bufs_and_sems[NBUF:2 * NBUF]
        ssem = bufs_and_sems[2 * NBUF:]
        wid = lax.axis_index("s") * NC + lax.axis_index("c")
        base = wid * n_per_w

        pltpu.sync_copy(fl_hbm.at[pl.ds(base, n_per_w)], idx_v)

        def prep_idx(c):
            off = (lax.shift_right_logical(base + c * CH, b_shift) * V)
            offv = jnp.full((16,), off, jnp.int32)
            for k in range(CH // 16):
                sl = pl.ds(c * CH + k * 16, 16)
                idx_v[sl] = idx_v[sl] + offv

        def g_src(c):
            return tab_hbm.at[idx_v.at[pl.ds(c * CH, CH)]]

        def s_dst(c):
            return out_hbm.at[pl.ds(base + c * CH, CH)]

        for q in range(NBUF):
            prep_idx(q)
            pltpu.async_copy(g_src(q), rows[q], gsem[q])

        def group_body(g, carry):
            c0 = g * NBUF
            for q in range(NBUF):
                c = c0 + q
                pltpu.make_async_copy(g_src(c), rows[q], gsem[q]).wait()
                pltpu.async_copy(rows[q], s_dst(c), ssem[q])
            for q in range(NBUF):
                c = c0 + q
                cn = c + NBUF
                pltpu.make_async_copy(rows[q], s_dst(c), ssem[q]).wait()

                @pl.when(cn < n_ch)
                def _():
                    prep_idx(cn)
                    pltpu.async_copy(g_src(cn), rows[q], gsem[q])

            return carry

        lax.fori_loop(0, n_groups, group_body, 0)

    return gather_kernel(tab_flat, fl_fm)


def _tc_linear_relu(emb3, w_bf16, bias, F, B, D, DOUT):
    BM = 256

    def mm_kernel(e_ref, w_ref, b_ref, o_ref):
        acc = jnp.zeros((BM, DOUT), jnp.float32)
        for k in range(F // 2):
            e2 = jnp.concatenate(
                [e_ref[2 * k].astype(jnp.bfloat16),
                 e_ref[2 * k + 1].astype(jnp.bfloat16)], axis=-1)
            acc = acc + jnp.dot(e2, w_ref[pl.ds(2 * k * D, 2 * D), :],
                                preferred_element_type=jnp.float32)
        if F % 2:
            acc = acc + jnp.dot(e_ref[F - 1].astype(jnp.bfloat16),
                                w_ref[pl.ds((F - 1) * D, D), :],
                                preferred_element_type=jnp.float32)
        o_ref[...] = jnp.maximum(acc + b_ref[...], 0.0)

    return pl.pallas_call(
        mm_kernel,
        grid=(B // BM,),
        in_specs=[
            pl.BlockSpec((F, BM, D), lambda i: (0, i, 0)),
            pl.BlockSpec((F * D, DOUT), lambda i: (0, 0)),
            pl.BlockSpec((1, DOUT), lambda i: (0, 0)),
        ],
        out_specs=pl.BlockSpec((BM, DOUT), lambda i: (i, 0)),
        out_shape=jax.ShapeDtypeStruct((B, DOUT), jnp.float32),
    )(emb3, w_bf16, bias)


def kernel(f_list, tables, W, b):
    F, V, D = tables.shape
    B = f_list.shape[0]
    DOUT = W.shape[1]
    tab_flat = tables.reshape(F * V, D)
    fl_t = f_list.T.astype(jnp.int32)
    w_bf16 = W.astype(jnp.bfloat16)
    bias2 = b.reshape(1, DOUT)
    bc = B // NCHUNK
    outs = []
    for c in range(NCHUNK):
        fl_c = fl_t[:, c * bc:(c + 1) * bc].reshape(F * bc)
        emb_c = _sc_gather(tab_flat, fl_c, F, V, D, bc)
        outs.append(_tc_linear_relu(emb_c.reshape(F, bc, D), w_bf16, bias2,
                                    F, bc, D, DOUT))
    return jnp.concatenate(outs, axis=0)
